# baseline (speedup 1.0000x reference)
import functools

import jax
import jax.numpy as jnp
from jax import lax
from jax.experimental import pallas as pl
from jax.experimental.pallas import tpu as pltpu
from jax.experimental.pallas import tpu_sc as plsc

_DIM = 512
_N_CLASS = 2
_N = 32768
_ROWS = 256
_LANES = 128
_K = int(0.2 * _N)
_NSEL = 2 * _K
_NW = 32
_PAD_SEL = 13312
_B_PER_W = _PAD_SEL // _NW


def _row_col_iota():
    r = lax.broadcasted_iota(jnp.int32, (_ROWS, _LANES), 0)
    c = lax.broadcasted_iota(jnp.int32, (_ROWS, _LANES), 1)
    return r, c


def _partner(x, d, r, c):
    if d < _LANES:
        lo = (c & d) == 0
        return jnp.where(lo, jnp.roll(x, -d, axis=1), jnp.roll(x, d, axis=1))
    dr = d // _LANES
    lo = (r & dr) == 0
    return jnp.where(lo, jnp.roll(x, -dr, axis=0), jnp.roll(x, dr, axis=0))


def _i_bit_zero(d, r, c):
    if d < _LANES:
        return (c & d) == 0
    return (r & (d // _LANES)) == 0


def _shift_down(x, c):
    a = jnp.roll(x, -1, axis=1)
    b = jnp.roll(a, -1, axis=0)
    return jnp.where(c == _LANES - 1, b, a)


def _shift_up(x, c):
    a = jnp.roll(x, 1, axis=1)
    b = jnp.roll(a, 1, axis=0)
    return jnp.where(c == 0, b, a)


_TR = 8
_NT = _ROWS // _TR
_TILE_N = _TR * _LANES


def _tile_iota():
    r = lax.broadcasted_iota(jnp.int32, (_TR, _LANES), 0)
    c = lax.broadcasted_iota(jnp.int32, (_TR, _LANES), 1)
    return r, c


def _roll2(x, sh, axis):
    n = x.shape[axis]
    sh %= n
    if sh == 0:
        return x
    if axis == 0:
        return jnp.concatenate([x[n - sh:, :], x[:n - sh, :]], axis=0)
    return jnp.concatenate([x[:, n - sh:], x[:, :n - sh]], axis=1)


def _tile_partner(x, d, r, c):
    if d < _LANES:
        lo = (c & d) == 0
        return jnp.where(lo, _roll2(x, -d, 1), _roll2(x, d, 1))
    dr = d // _LANES
    lo = (r & dr) == 0
    return jnp.where(lo, _roll2(x, -dr, 0), _roll2(x, dr, 0))


def _tile_substage(kb, ib, d, up, r, c):
    pk = _tile_partner(kb, d, r, c)
    pv = _tile_partner(ib, d, r, c)
    if d < _LANES:
        i_lower = (c & d) == 0
    else:
        i_lower = (r & (d // _LANES)) == 0
    me_first = (kb < pk) | ((kb == pk) & (ib < pv))
    take = me_first != (i_lower == up)
    return jnp.where(take, pk, kb), jnp.where(take, pv, ib)


def _full_substage(ku, iv, d, kbit, r, c):
    pk = _partner(ku, d, r, c)
    pv = _partner(iv, d, r, c)
    i_lower = _i_bit_zero(d, r, c)
    up = _i_bit_zero(kbit, r, c)
    me_first = (ku < pk) | ((ku == pk) & (iv < pv))
    take = me_first != (i_lower == up)
    return jnp.where(take, pk, ku), jnp.where(take, pv, iv)


def _sort_kernel(a_ref, asc_ref, pos_ref, key_s, idx_s):
    r, c = _row_col_iota()
    u = lax.bitcast_convert_type(a_ref[...], jnp.int32)
    key_s[...] = u ^ ((u >> 31) & jnp.int32(0x7FFFFFFF))
    idx_s[...] = r * _LANES + c

    rt, ct = _tile_iota()

    def phase1(t, _):
        kb = key_s[pl.ds(t * _TR, _TR), :]
        ib = idx_s[pl.ds(t * _TR, _TR), :]
        gi = (t * _TR + rt) * _LANES + ct
        for k_exp in range(1, 11):
            kbit = 1 << k_exp
            up = (gi & kbit) == 0
            for j_exp in range(k_exp - 1, -1, -1):
                kb, ib = _tile_substage(kb, ib, 1 << j_exp, up, rt, ct)
        key_s[pl.ds(t * _TR, _TR), :] = kb
        idx_s[pl.ds(t * _TR, _TR), :] = ib
        return 0

    lax.fori_loop(0, _NT, phase1, 0)

    for k_exp in range(11, 16):
        kbit = 1 << k_exp
        ku = key_s[...]
        iv = idx_s[...]
        for j_exp in range(k_exp - 1, 9, -1):
            ku, iv = _full_substage(ku, iv, 1 << j_exp, kbit, r, c)
        key_s[...] = ku
        idx_s[...] = iv

        def tail(t, _, k_exp=k_exp, kbit=kbit):
            kb = key_s[pl.ds(t * _TR, _TR), :]
            ib = idx_s[pl.ds(t * _TR, _TR), :]
            gi = (t * _TR + rt) * _LANES + ct
            up = (gi & kbit) == 0
            for j_exp in range(9, -1, -1):
                kb, ib = _tile_substage(kb, ib, 1 << j_exp, up, rt, ct)
            key_s[pl.ds(t * _TR, _TR), :] = kb
            idx_s[pl.ds(t * _TR, _TR), :] = ib
            return 0

        lax.fori_loop(0, _NT, tail, 0)

    ku = key_s[...]
    iv = idx_s[...]
    asc_ref[...] = iv

    kf = ku
    vf = iv
    for p in (0, 1, 0, 1, 0):
        down_k = _shift_down(kf, c)
        down_v = _shift_down(vf, c)
        up_k = _shift_up(kf, c)
        up_v = _shift_up(vf, c)
        i_lower = (c & 1) == p
        pk = jnp.where(i_lower, down_k, up_k)
        pv = jnp.where(i_lower, down_v, up_v)
        me_first = (kf < pk) | ((kf == pk) & (vf > pv))
        take = me_first != i_lower
        if p == 1:
            first = (r == 0) & (c == 0)
            last = (r == _ROWS - 1) & (c == _LANES - 1)
            take = take & ~first & ~last
        kf = jnp.where(take, pk, kf)
        vf = jnp.where(take, pv, vf)

    pos_ref[...] = vf


def _matmul_kernel(h_ref, w_ref, b_ref, out_ref):
    out_ref[...] = jnp.dot(h_ref[...], w_ref[...],
                           preferred_element_type=jnp.float32) + b_ref[...]


def _make_sc_gather_softmax():
    mesh = plsc.VectorSubcoreMesh(core_axis_name="c", subcore_axis_name="s")

    @functools.partial(
        pl.kernel,
        mesh=mesh,
        compiler_params=pltpu.CompilerParams(needs_layout_passes=False),
        out_type=(
            jax.ShapeDtypeStruct((2 * _PAD_SEL,), jnp.float32),
            jax.ShapeDtypeStruct((2 * _PAD_SEL,), jnp.float32),
        ),
        scratch_types=[
            pltpu.VMEM((_B_PER_W,), jnp.int32),
            pltpu.VMEM((2 * _N,), jnp.float32),
            pltpu.VMEM((2 * _B_PER_W,), jnp.float32),
            pltpu.VMEM((2 * _B_PER_W,), jnp.float32),
        ],
    )
    def gather_softmax(ids_hbm, table_hbm, un_hbm, sm_hbm,
                       idx_v, table_v, un_v, sm_v):
        wid = lax.axis_index("s") * 2 + lax.axis_index("c")
        base = wid * _B_PER_W
        pltpu.sync_copy(ids_hbm.at[pl.ds(base, _B_PER_W)], idx_v)
        pltpu.sync_copy(table_hbm, table_v)
        lane = lax.iota(jnp.int32, 16)
        for j in range(_B_PER_W // 16):
            idx16 = idx_v[pl.ds(j * 16, 16)]
            flat = idx16 * 2
            x0 = plsc.load_gather(table_v, [flat])
            x1 = plsc.load_gather(table_v, [flat + 1])
            m = jnp.maximum(x0, x1)
            e0 = jnp.exp(x0 - m)
            e1 = jnp.exp(x1 - m)
            s = e0 + e1
            p0 = e0 / s
            p1 = e1 / s
            out_pos = (lane + j * 16) * 2
            plsc.store_scatter(un_v, [out_pos], x0)
            plsc.store_scatter(un_v, [out_pos + 1], x1)
            plsc.store_scatter(sm_v, [out_pos], p0)
            plsc.store_scatter(sm_v, [out_pos + 1], p1)
        pltpu.sync_copy(un_v, un_hbm.at[pl.ds(base * 2, 2 * _B_PER_W)])
        pltpu.sync_copy(sm_v, sm_hbm.at[pl.ds(base * 2, 2 * _B_PER_W)])

    return gather_softmax


def kernel(bag_label, h, A, W, b):
    a_i = jnp.take(A[:, 0, :], bag_label, axis=1)

    asc_idx, pos_arr = pl.pallas_call(
        _sort_kernel,
        out_shape=(
            jax.ShapeDtypeStruct((_ROWS, _LANES), jnp.int32),
            jax.ShapeDtypeStruct((_ROWS, _LANES), jnp.int32),
        ),
        scratch_shapes=[
            pltpu.VMEM((_ROWS, _LANES), jnp.int32),
            pltpu.VMEM((_ROWS, _LANES), jnp.int32),
        ],
    )(a_i.reshape(_ROWS, _LANES))

    blk = 2048
    hw = pl.pallas_call(
        _matmul_kernel,
        grid=(_N // blk,),
        in_specs=[
            pl.BlockSpec((blk, _DIM), lambda i: (i, 0)),
            pl.BlockSpec((_DIM, _N_CLASS), lambda i: (0, 0)),
            pl.BlockSpec((1, _N_CLASS), lambda i: (0, 0)),
        ],
        out_specs=pl.BlockSpec((blk, _N_CLASS), lambda i: (i, 0)),
        out_shape=jax.ShapeDtypeStruct((_N, _N_CLASS), jnp.float32),
    )(h, W, b.reshape(1, _N_CLASS))

    neg_ids = asc_idx.reshape(-1)[:_K]
    pos_ids = pos_arr.reshape(-1)[::-1][:_K]
    ids = jnp.concatenate(
        [pos_ids, neg_ids, jnp.zeros((_PAD_SEL - _NSEL,), jnp.int32)])

    un_flat, sm_flat = _make_sc_gather_softmax()(ids, hw.reshape(-1))

    logits_unnorm = un_flat.reshape(_PAD_SEL, 2)[:_NSEL]
    logits = sm_flat.reshape(_PAD_SEL, 2)[:_NSEL]
    ins_labels = jnp.concatenate(
        [jnp.ones((_K,), jnp.int32), jnp.zeros((_K,), jnp.int32)])
    return (ins_labels, logits_unnorm, logits)


# speedup vs baseline: 2.5438x; 2.5438x over previous
import functools

import jax
import jax.numpy as jnp
from jax import lax
from jax.experimental import pallas as pl
from jax.experimental.pallas import tpu as pltpu
from jax.experimental.pallas import tpu_sc as plsc

_DIM = 512
_N_CLASS = 2
_N = 32768
_ROWS = 256
_LANES = 128
_K = int(0.2 * _N)
_NSEL = 2 * _K
_NW = 32
_PAD_SEL = 13312
_B_PER_W = _PAD_SEL // _NW


def _row_col_iota():
    r = lax.broadcasted_iota(jnp.int32, (_ROWS, _LANES), 0)
    c = lax.broadcasted_iota(jnp.int32, (_ROWS, _LANES), 1)
    return r, c


def _partner(x, d, r, c):
    if d < _LANES:
        lo = (c & d) == 0
        return jnp.where(lo, jnp.roll(x, -d, axis=1), jnp.roll(x, d, axis=1))
    dr = d // _LANES
    lo = (r & dr) == 0
    return jnp.where(lo, jnp.roll(x, -dr, axis=0), jnp.roll(x, dr, axis=0))


def _i_bit_zero(d, r, c):
    if d < _LANES:
        return (c & d) == 0
    return (r & (d // _LANES)) == 0


def _shift_down(x, c):
    a = jnp.roll(x, -1, axis=1)
    b = jnp.roll(a, -1, axis=0)
    return jnp.where(c == _LANES - 1, b, a)


def _shift_up(x, c):
    a = jnp.roll(x, 1, axis=1)
    b = jnp.roll(a, 1, axis=0)
    return jnp.where(c == 0, b, a)


def _sort_kernel(a_ref, asc_ref, pos_ref):
    r, c = _row_col_iota()
    u = lax.bitcast_convert_type(a_ref[...], jnp.int32)
    ku = u ^ ((u >> 31) & jnp.int32(0x7FFFFFFF))
    iv = r * _LANES + c

    for k_exp in range(1, 16):
        kbit = 1 << k_exp
        up = _i_bit_zero(kbit, r, c)
        for j_exp in range(k_exp - 1, -1, -1):
            d = 1 << j_exp
            pk = _partner(ku, d, r, c)
            pv = _partner(iv, d, r, c)
            i_lower = _i_bit_zero(d, r, c)
            me_first = (ku < pk) | ((ku == pk) & (iv < pv))
            take = me_first != (i_lower == up)
            ku = jnp.where(take, pk, ku)
            iv = jnp.where(take, pv, iv)

    asc_ref[...] = iv

    kf = ku
    vf = iv
    for p in (0, 1, 0, 1, 0):
        down_k = _shift_down(kf, c)
        down_v = _shift_down(vf, c)
        up_k = _shift_up(kf, c)
        up_v = _shift_up(vf, c)
        i_lower = (c & 1) == p
        pk = jnp.where(i_lower, down_k, up_k)
        pv = jnp.where(i_lower, down_v, up_v)
        me_first = (kf < pk) | ((kf == pk) & (vf > pv))
        take = me_first != i_lower
        if p == 1:
            first = (r == 0) & (c == 0)
            last = (r == _ROWS - 1) & (c == _LANES - 1)
            take = take & ~first & ~last
        kf = jnp.where(take, pk, kf)
        vf = jnp.where(take, pv, vf)

    pos_ref[...] = vf


def _matmul_kernel(h_ref, w_ref, b_ref, out_ref):
    out_ref[...] = jnp.dot(h_ref[...], w_ref[...],
                           preferred_element_type=jnp.float32) + b_ref[...]


def _make_sc_gather_softmax():
    mesh = plsc.VectorSubcoreMesh(core_axis_name="c", subcore_axis_name="s")

    @functools.partial(
        pl.kernel,
        mesh=mesh,
        compiler_params=pltpu.CompilerParams(needs_layout_passes=False),
        out_type=(
            jax.ShapeDtypeStruct((2 * _PAD_SEL,), jnp.float32),
            jax.ShapeDtypeStruct((2 * _PAD_SEL,), jnp.float32),
        ),
        scratch_types=[
            pltpu.VMEM((_B_PER_W,), jnp.int32),
            pltpu.VMEM((_B_PER_W + 16,), jnp.int32),
            pltpu.VMEM((2 * _B_PER_W,), jnp.int32),
            pltpu.VMEM((2 * _B_PER_W,), jnp.float32),
            pltpu.VMEM((2 * _B_PER_W,), jnp.float32),
            pltpu.VMEM((2 * _B_PER_W,), jnp.float32),
            pltpu.SemaphoreType.DMA,
        ],
    )
    def gather_softmax(pos_hbm, asc_hbm, table_hbm, un_hbm, sm_hbm,
                       pos_v, neg_v, ids_v, rows_v, un_v, sm_v, sem):
        wid = lax.axis_index("s") * 2 + lax.axis_index("c")
        base = pl.multiple_of(wid * _B_PER_W, 8)
        pltpu.sync_copy(
            pos_hbm.at[pl.ds(pl.multiple_of(_N - _B_PER_W - base, 8),
                             _B_PER_W)], pos_v)
        neg_start = pl.multiple_of(
            jnp.maximum(base - (_K + 7), 0), 8)
        pltpu.sync_copy(asc_hbm.at[pl.ds(neg_start, _B_PER_W + 16)], neg_v)
        lane = lax.iota(jnp.int32, 16)
        nchunk = _B_PER_W // 16
        for j in range(nchunk):
            q = base + j * 16 + lane
            pidx = (_B_PER_W - 1) - (j * 16 + lane)
            nidx = jnp.clip(q - _K - neg_start, 0, _B_PER_W + 15)
            idp = plsc.load_gather(pos_v, [pidx])
            idn = plsc.load_gather(neg_v, [nidx])
            ids16 = jnp.where(q < _K, idp, idn)
            ids16 = jnp.where(q < _NSEL, ids16, 0)
            flat2 = ids16 * 2
            out16 = (lane + j * 16) * 2
            plsc.store_scatter(ids_v, [out16], flat2)
            plsc.store_scatter(ids_v, [out16 + 1], flat2 + 1)
        pltpu.async_copy(table_hbm.at[ids_v], rows_v, sem).wait()
        for j in range(nchunk):
            row = lane + j * 16
            x0 = plsc.load_gather(rows_v, [row * 2])
            x1 = plsc.load_gather(rows_v, [row * 2 + 1])
            m = jnp.maximum(x0, x1)
            e0 = jnp.exp(x0 - m)
            e1 = jnp.exp(x1 - m)
            s = e0 + e1
            p0 = row * 2
            plsc.store_scatter(un_v, [p0], x0)
            plsc.store_scatter(un_v, [p0 + 1], x1)
            plsc.store_scatter(sm_v, [p0], e0 / s)
            plsc.store_scatter(sm_v, [p0 + 1], e1 / s)
        obase = pl.multiple_of(base * 2, 8)
        pltpu.sync_copy(un_v, un_hbm.at[pl.ds(obase, 2 * _B_PER_W)])
        pltpu.sync_copy(sm_v, sm_hbm.at[pl.ds(obase, 2 * _B_PER_W)])

    return gather_softmax


def kernel(bag_label, h, A, W, b):
    a_i = jnp.take(A[:, 0, :], bag_label, axis=1)

    asc_idx, pos_arr = pl.pallas_call(
        _sort_kernel,
        out_shape=(
            jax.ShapeDtypeStruct((_ROWS, _LANES), jnp.int32),
            jax.ShapeDtypeStruct((_ROWS, _LANES), jnp.int32),
        ),
    )(a_i.reshape(_ROWS, _LANES))

    blk = 2048
    hw = pl.pallas_call(
        _matmul_kernel,
        grid=(_N // blk,),
        in_specs=[
            pl.BlockSpec((blk, _DIM), lambda i: (i, 0)),
            pl.BlockSpec((_DIM, _N_CLASS), lambda i: (0, 0)),
            pl.BlockSpec((1, _N_CLASS), lambda i: (0, 0)),
        ],
        out_specs=pl.BlockSpec((blk, _N_CLASS), lambda i: (i, 0)),
        out_shape=jax.ShapeDtypeStruct((_N, _N_CLASS), jnp.float32),
    )(h, W, b.reshape(1, _N_CLASS))

    un_flat, sm_flat = _make_sc_gather_softmax()(
        pos_arr.reshape(-1), asc_idx.reshape(-1), hw.reshape(-1))

    logits_unnorm = un_flat.reshape(_PAD_SEL, 2)[:_NSEL]
    logits = sm_flat.reshape(_PAD_SEL, 2)[:_NSEL]
    ins_labels = jnp.concatenate(
        [jnp.ones((_K,), jnp.int32), jnp.zeros((_K,), jnp.int32)])
    return (ins_labels, logits_unnorm, logits)


# speedup vs baseline: 2.8944x; 1.1378x over previous
import functools

import jax
import jax.numpy as jnp
from jax import lax
from jax.experimental import pallas as pl
from jax.experimental.pallas import tpu as pltpu
from jax.experimental.pallas import tpu_sc as plsc

_DIM = 512
_N_CLASS = 2
_N = 32768
_ROWS = 256
_LANES = 128
_K = int(0.2 * _N)
_NSEL = 2 * _K
_NW = 32
_PAD_SEL = 13312
_B_PER_W = _PAD_SEL // _NW


def _row_col_iota():
    r = lax.broadcasted_iota(jnp.int32, (_ROWS, _LANES), 0)
    c = lax.broadcasted_iota(jnp.int32, (_ROWS, _LANES), 1)
    return r, c


def _partner(x, d, r, c):
    if d < _LANES:
        lo = (c & d) == 0
        return jnp.where(lo, jnp.roll(x, -d, axis=1), jnp.roll(x, d, axis=1))
    dr = d // _LANES
    lo = (r & dr) == 0
    return jnp.where(lo, jnp.roll(x, -dr, axis=0), jnp.roll(x, dr, axis=0))


def _i_bit_zero(d, r, c):
    if d < _LANES:
        return (c & d) == 0
    return (r & (d // _LANES)) == 0


def _shift_down(x, c):
    a = jnp.roll(x, -1, axis=1)
    b = jnp.roll(a, -1, axis=0)
    return jnp.where(c == _LANES - 1, b, a)


def _shift_up(x, c):
    a = jnp.roll(x, 1, axis=1)
    b = jnp.roll(a, 1, axis=0)
    return jnp.where(c == 0, b, a)


def _sort_kernel(a_ref, asc_ref, pos_ref):
    r, c = _row_col_iota()
    u = lax.bitcast_convert_type(a_ref[...], jnp.int32)
    ku = u ^ ((u >> 31) & jnp.int32(0x7FFFFFFF))
    iv = r * _LANES + c

    for k_exp in range(1, 16):
        kbit = 1 << k_exp
        up = _i_bit_zero(kbit, r, c)
        for j_exp in range(k_exp - 1, -1, -1):
            d = 1 << j_exp
            pk = _partner(ku, d, r, c)
            pv = _partner(iv, d, r, c)
            i_lower = _i_bit_zero(d, r, c)
            me_first = (ku < pk) | ((ku == pk) & (iv < pv))
            take = me_first != (i_lower == up)
            ku = jnp.where(take, pk, ku)
            iv = jnp.where(take, pv, iv)

    asc_ref[...] = iv

    kf = ku
    vf = iv
    for p in (0, 1, 0, 1, 0):
        down_k = _shift_down(kf, c)
        down_v = _shift_down(vf, c)
        up_k = _shift_up(kf, c)
        up_v = _shift_up(vf, c)
        i_lower = (c & 1) == p
        pk = jnp.where(i_lower, down_k, up_k)
        pv = jnp.where(i_lower, down_v, up_v)
        me_first = (kf < pk) | ((kf == pk) & (vf > pv))
        take = me_first != i_lower
        if p == 1:
            first = (r == 0) & (c == 0)
            last = (r == _ROWS - 1) & (c == _LANES - 1)
            take = take & ~first & ~last
        kf = jnp.where(take, pk, kf)
        vf = jnp.where(take, pv, vf)

    pos_ref[...] = vf


def _matmul_kernel(h_ref, w_ref, b_ref, out_ref):
    out_ref[...] = lax.dot_general(
        w_ref[...], h_ref[...],
        (((0,), (1,)), ((), ())),
        preferred_element_type=jnp.float32) + b_ref[...]


def _make_sc_gather_softmax():
    mesh = plsc.VectorSubcoreMesh(core_axis_name="c", subcore_axis_name="s")

    @functools.partial(
        pl.kernel,
        mesh=mesh,
        compiler_params=pltpu.CompilerParams(needs_layout_passes=False),
        out_type=(
            jax.ShapeDtypeStruct((2 * _PAD_SEL,), jnp.float32),
            jax.ShapeDtypeStruct((2 * _PAD_SEL,), jnp.float32),
        ),
        scratch_types=[
            pltpu.VMEM((_B_PER_W,), jnp.int32),
            pltpu.VMEM((_B_PER_W + 16,), jnp.int32),
            pltpu.VMEM((2 * _B_PER_W,), jnp.int32),
            pltpu.VMEM((2 * _B_PER_W,), jnp.float32),
            pltpu.VMEM((2 * _B_PER_W,), jnp.float32),
            pltpu.VMEM((2 * _B_PER_W,), jnp.float32),
            pltpu.SemaphoreType.DMA,
        ],
    )
    def gather_softmax(pos_hbm, asc_hbm, table_hbm, un_hbm, sm_hbm,
                       pos_v, neg_v, ids_v, rows_v, un_v, sm_v, sem):
        wid = lax.axis_index("s") * 2 + lax.axis_index("c")
        base = pl.multiple_of(wid * _B_PER_W, 8)
        pltpu.sync_copy(
            pos_hbm.at[pl.ds(pl.multiple_of(_N - _B_PER_W - base, 8),
                             _B_PER_W)], pos_v)
        neg_start = pl.multiple_of(
            jnp.maximum(base - (_K + 7), 0), 8)
        pltpu.sync_copy(asc_hbm.at[pl.ds(neg_start, _B_PER_W + 16)], neg_v)
        lane = lax.iota(jnp.int32, 16)
        nchunk = _B_PER_W // 16
        for j in range(nchunk):
            q = base + j * 16 + lane
            pidx = (_B_PER_W - 1) - (j * 16 + lane)
            nidx = jnp.clip(q - _K - neg_start, 0, _B_PER_W + 15)
            idp = plsc.load_gather(pos_v, [pidx])
            idn = plsc.load_gather(neg_v, [nidx])
            ids16 = jnp.where(q < _K, idp, idn)
            ids16 = jnp.where(q < _NSEL, ids16, 0)
            out16 = (lane + j * 16) * 2
            plsc.store_scatter(ids_v, [out16], ids16)
            plsc.store_scatter(ids_v, [out16 + 1], ids16 + _N)
        pltpu.async_copy(table_hbm.at[ids_v], rows_v, sem).wait()
        for j in range(nchunk):
            row = lane + j * 16
            x0 = plsc.load_gather(rows_v, [row * 2])
            x1 = plsc.load_gather(rows_v, [row * 2 + 1])
            m = jnp.maximum(x0, x1)
            e0 = jnp.exp(x0 - m)
            e1 = jnp.exp(x1 - m)
            s = e0 + e1
            p0 = row * 2
            plsc.store_scatter(un_v, [p0], x0)
            plsc.store_scatter(un_v, [p0 + 1], x1)
            plsc.store_scatter(sm_v, [p0], e0 / s)
            plsc.store_scatter(sm_v, [p0 + 1], e1 / s)
        obase = pl.multiple_of(base * 2, 8)
        pltpu.sync_copy(un_v, un_hbm.at[pl.ds(obase, 2 * _B_PER_W)])
        pltpu.sync_copy(sm_v, sm_hbm.at[pl.ds(obase, 2 * _B_PER_W)])

    return gather_softmax


def kernel(bag_label, h, A, W, b):
    a_i = jnp.take(A[:, 0, :], bag_label, axis=1)

    asc_idx, pos_arr = pl.pallas_call(
        _sort_kernel,
        out_shape=(
            jax.ShapeDtypeStruct((_ROWS, _LANES), jnp.int32),
            jax.ShapeDtypeStruct((_ROWS, _LANES), jnp.int32),
        ),
    )(a_i.reshape(_ROWS, _LANES))

    blk = 2048
    hw = pl.pallas_call(
        _matmul_kernel,
        grid=(_N // blk,),
        in_specs=[
            pl.BlockSpec((blk, _DIM), lambda i: (i, 0)),
            pl.BlockSpec((_DIM, _N_CLASS), lambda i: (0, 0)),
            pl.BlockSpec((_N_CLASS, 1), lambda i: (0, 0)),
        ],
        out_specs=pl.BlockSpec((_N_CLASS, blk), lambda i: (0, i)),
        out_shape=jax.ShapeDtypeStruct((_N_CLASS, _N), jnp.float32),
    )(h, W, b.reshape(_N_CLASS, 1))

    un_flat, sm_flat = _make_sc_gather_softmax()(
        pos_arr.reshape(-1), asc_idx.reshape(-1), hw.reshape(-1))

    logits_unnorm = un_flat.reshape(_PAD_SEL, 2)[:_NSEL]
    logits = sm_flat.reshape(_PAD_SEL, 2)[:_NSEL]
    ins_labels = jnp.concatenate(
        [jnp.ones((_K,), jnp.int32), jnp.zeros((_K,), jnp.int32)])
    return (ins_labels, logits_unnorm, logits)


# speedup vs baseline: 2.8960x; 1.0006x over previous
import functools

import jax
import jax.numpy as jnp
from jax import lax
from jax.experimental import pallas as pl
from jax.experimental.pallas import tpu as pltpu
from jax.experimental.pallas import tpu_sc as plsc

_DIM = 512
_N_CLASS = 2
_N = 32768
_ROWS = 256
_LANES = 128
_K = int(0.2 * _N)
_NSEL = 2 * _K
_NW = 32
_PAD_SEL = 13312
_B_PER_W = _PAD_SEL // _NW


def _row_col_iota():
    r = lax.broadcasted_iota(jnp.int32, (_ROWS, _LANES), 0)
    c = lax.broadcasted_iota(jnp.int32, (_ROWS, _LANES), 1)
    return r, c


def _partner(x, d, r, c):
    if d < _LANES:
        lo = (c & d) == 0
        return jnp.where(lo, jnp.roll(x, -d, axis=1), jnp.roll(x, d, axis=1))
    dr = d // _LANES
    lo = (r & dr) == 0
    return jnp.where(lo, jnp.roll(x, -dr, axis=0), jnp.roll(x, dr, axis=0))


def _i_bit_zero(d, r, c):
    if d < _LANES:
        return (c & d) == 0
    return (r & (d // _LANES)) == 0


def _shift_down(x, c):
    a = jnp.roll(x, -1, axis=1)
    b = jnp.roll(a, -1, axis=0)
    return jnp.where(c == _LANES - 1, b, a)


def _shift_up(x, c):
    a = jnp.roll(x, 1, axis=1)
    b = jnp.roll(a, 1, axis=0)
    return jnp.where(c == 0, b, a)


def _sort_kernel(a_ref, asc_ref, pos_ref):
    r, c = _row_col_iota()
    u = lax.bitcast_convert_type(a_ref[...], jnp.int32)
    ku = u ^ ((u >> 31) & jnp.int32(0x7FFFFFFF))
    iv = r * _LANES + c

    for k_exp in range(1, 16):
        kbit = 1 << k_exp
        up = _i_bit_zero(kbit, r, c)
        for j_exp in range(k_exp - 1, -1, -1):
            d = 1 << j_exp
            pk = _partner(ku, d, r, c)
            pv = _partner(iv, d, r, c)
            i_lower = _i_bit_zero(d, r, c)
            me_first = (ku < pk) | ((ku == pk) & (iv < pv))
            take = me_first != (i_lower == up)
            ku = jnp.where(take, pk, ku)
            iv = jnp.where(take, pv, iv)

    asc_ref[...] = iv

    kf = ku
    vf = iv
    for p in (0, 1, 0, 1, 0):
        down_k = _shift_down(kf, c)
        down_v = _shift_down(vf, c)
        up_k = _shift_up(kf, c)
        up_v = _shift_up(vf, c)
        i_lower = (c & 1) == p
        pk = jnp.where(i_lower, down_k, up_k)
        pv = jnp.where(i_lower, down_v, up_v)
        me_first = (kf < pk) | ((kf == pk) & (vf > pv))
        take = me_first != i_lower
        if p == 1:
            first = (r == 0) & (c == 0)
            last = (r == _ROWS - 1) & (c == _LANES - 1)
            take = take & ~first & ~last
        kf = jnp.where(take, pk, kf)
        vf = jnp.where(take, pv, vf)

    pos_ref[...] = vf


def _matmul_kernel(h_ref, w_ref, b_ref, out_ref):
    out_ref[...] = lax.dot_general(
        w_ref[...], h_ref[...],
        (((0,), (1,)), ((), ())),
        preferred_element_type=jnp.float32) + b_ref[...]


def _make_sc_gather_softmax():
    mesh = plsc.VectorSubcoreMesh(core_axis_name="c", subcore_axis_name="s")

    @functools.partial(
        pl.kernel,
        mesh=mesh,
        compiler_params=pltpu.CompilerParams(needs_layout_passes=False),
        out_type=(
            jax.ShapeDtypeStruct((2 * _NSEL,), jnp.float32),
            jax.ShapeDtypeStruct((2 * _NSEL,), jnp.float32),
        ),
        scratch_types=[
            pltpu.VMEM((_B_PER_W,), jnp.int32),
            pltpu.VMEM((_B_PER_W + 16,), jnp.int32),
            pltpu.VMEM((2 * _B_PER_W,), jnp.int32),
            pltpu.VMEM((2 * _B_PER_W,), jnp.float32),
            pltpu.VMEM((2 * _B_PER_W,), jnp.float32),
            pltpu.VMEM((2 * _B_PER_W,), jnp.float32),
            pltpu.SemaphoreType.DMA,
        ],
    )
    def gather_softmax(pos_hbm, asc_hbm, table_hbm, un_hbm, sm_hbm,
                       pos_v, neg_v, ids_v, rows_v, un_v, sm_v, sem):
        wid = lax.axis_index("s") * 2 + lax.axis_index("c")
        base = pl.multiple_of(wid * _B_PER_W, 8)
        pltpu.sync_copy(
            pos_hbm.at[pl.ds(pl.multiple_of(_N - _B_PER_W - base, 8),
                             _B_PER_W)], pos_v)
        neg_start = pl.multiple_of(
            jnp.maximum(base - (_K + 7), 0), 8)
        pltpu.sync_copy(asc_hbm.at[pl.ds(neg_start, _B_PER_W + 16)], neg_v)
        lane = lax.iota(jnp.int32, 16)
        nchunk = _B_PER_W // 16
        for j in range(nchunk):
            q = base + j * 16 + lane
            pidx = (_B_PER_W - 1) - (j * 16 + lane)
            nidx = jnp.clip(q - _K - neg_start, 0, _B_PER_W + 15)
            idp = plsc.load_gather(pos_v, [pidx])
            idn = plsc.load_gather(neg_v, [nidx])
            ids16 = jnp.where(q < _K, idp, idn)
            ids16 = jnp.where(q < _NSEL, ids16, 0)
            out16 = (lane + j * 16) * 2
            plsc.store_scatter(ids_v, [out16], ids16)
            plsc.store_scatter(ids_v, [out16 + 1], ids16 + _N)
        pltpu.async_copy(table_hbm.at[ids_v], rows_v, sem).wait()
        for j in range(nchunk):
            row = lane + j * 16
            x0 = plsc.load_gather(rows_v, [row * 2])
            x1 = plsc.load_gather(rows_v, [row * 2 + 1])
            m = jnp.maximum(x0, x1)
            e0 = jnp.exp(x0 - m)
            e1 = jnp.exp(x1 - m)
            s = e0 + e1
            p0 = row * 2
            plsc.store_scatter(un_v, [p0], x0)
            plsc.store_scatter(un_v, [p0 + 1], x1)
            plsc.store_scatter(sm_v, [p0], e0 / s)
            plsc.store_scatter(sm_v, [p0 + 1], e1 / s)
        obase = pl.multiple_of(base * 2, 8)

        @pl.when(wid < _NW - 1)
        def _():
            pltpu.sync_copy(un_v, un_hbm.at[pl.ds(obase, 2 * _B_PER_W)])
            pltpu.sync_copy(sm_v, sm_hbm.at[pl.ds(obase, 2 * _B_PER_W)])

        last = 2 * _NSEL - (_NW - 1) * 2 * _B_PER_W

        @pl.when(wid == _NW - 1)
        def _():
            pltpu.sync_copy(un_v.at[pl.ds(0, last)],
                            un_hbm.at[pl.ds(obase, last)])
            pltpu.sync_copy(sm_v.at[pl.ds(0, last)],
                            sm_hbm.at[pl.ds(obase, last)])

    return gather_softmax


def kernel(bag_label, h, A, W, b):
    a_i = jnp.take(A[:, 0, :], bag_label, axis=1)

    asc_idx, pos_arr = pl.pallas_call(
        _sort_kernel,
        out_shape=(
            jax.ShapeDtypeStruct((_ROWS, _LANES), jnp.int32),
            jax.ShapeDtypeStruct((_ROWS, _LANES), jnp.int32),
        ),
    )(a_i.reshape(_ROWS, _LANES))

    blk = 2048
    hw = pl.pallas_call(
        _matmul_kernel,
        grid=(_N // blk,),
        in_specs=[
            pl.BlockSpec((blk, _DIM), lambda i: (i, 0)),
            pl.BlockSpec((_DIM, _N_CLASS), lambda i: (0, 0)),
            pl.BlockSpec((_N_CLASS, 1), lambda i: (0, 0)),
        ],
        out_specs=pl.BlockSpec((_N_CLASS, blk), lambda i: (0, i)),
        out_shape=jax.ShapeDtypeStruct((_N_CLASS, _N), jnp.float32),
    )(h, W, b.reshape(_N_CLASS, 1))

    un_flat, sm_flat = _make_sc_gather_softmax()(
        pos_arr.reshape(-1), asc_idx.reshape(-1), hw.reshape(-1))

    logits_unnorm = un_flat.reshape(_NSEL, 2)
    logits = sm_flat.reshape(_NSEL, 2)
    ins_labels = jnp.concatenate(
        [jnp.ones((_K,), jnp.int32), jnp.zeros((_K,), jnp.int32)])
    return (ins_labels, logits_unnorm, logits)
